# baseline (speedup 1.0000x reference)
import functools

import jax
import jax.numpy as jnp
from jax import lax
from jax.experimental import pallas as pl
from jax.experimental.pallas import tpu as pltpu
from jax.experimental.pallas import tpu_sc as plsc

BATCH = 16384
D = 50
NC, NS, L = 2, 16, 16
NW = NC * NS
BPW = BATCH // NW
NBLK = BPW // L


@functools.partial(
    pl.kernel,
    out_type=jax.ShapeDtypeStruct((BATCH,), jnp.float32),
    mesh=plsc.VectorSubcoreMesh(core_axis_name="c", subcore_axis_name="s"),
    compiler_params=pltpu.CompilerParams(
        needs_layout_passes=False, use_tc_tiling_on_sc=False),
    scratch_types=[
        pltpu.VMEM((BPW, 2), jnp.int32),
        pltpu.VMEM((BPW, D), jnp.float32),
        pltpu.VMEM((BPW, D), jnp.float32),
        pltpu.VMEM((D, 1), jnp.float32),
        pltpu.VMEM((D + 1, L), jnp.float32),
        pltpu.VMEM((BPW,), jnp.float32),
    ] + [pltpu.SemaphoreType.DMA] * 8,
)
def _sc_dot(td_hbm, ue_hbm, me_hbm, w_hbm, b_hbm, out_hbm,
            idx2_v, urows_v, mrows_v, w_v, ws_v, out_v, *sems):
    wid = lax.axis_index("s") * NC + lax.axis_index("c")
    base = wid * BPW

    pltpu.sync_copy(td_hbm.at[pl.ds(base, BPW)], idx2_v)
    pltpu.sync_copy(w_hbm, w_v)
    for t in range(L):
        pltpu.sync_copy(b_hbm, ws_v.at[D, pl.ds(t, 1)])

    iota = lax.iota(jnp.int32, L)
    zeros = jnp.zeros((L,), jnp.int32)
    ones = jnp.ones((L,), jnp.int32)

    def fetch(blk, carry):
        r = blk * L + iota
        uvec = plsc.load_gather(idx2_v, [r, zeros])
        mvec = plsc.load_gather(idx2_v, [r, ones])
        for t in range(L):
            i = blk * L + t
            pltpu.async_copy(
                ue_hbm.at[pl.ds(uvec[t], 1)], urows_v.at[pl.ds(i, 1)],
                sems[t % 4])
            pltpu.async_copy(
                me_hbm.at[pl.ds(mvec[t], 1)], mrows_v.at[pl.ds(i, 1)],
                sems[4 + t % 4])
        return carry

    lax.fori_loop(0, NBLK, fetch, None)

    for d in range(D):
        ws_v[d, :] = plsc.load_gather(
            w_v, [jnp.full((L,), d, jnp.int32), zeros])

    for k in range(4):
        pltpu.make_async_copy(
            ue_hbm.at[pl.ds(0, BPW // 4)],
            urows_v.at[pl.ds(k * (BPW // 4), BPW // 4)], sems[k]).wait()
        pltpu.make_async_copy(
            me_hbm.at[pl.ds(0, BPW // 4)],
            mrows_v.at[pl.ds(k * (BPW // 4), BPW // 4)], sems[4 + k]).wait()

    bvec = ws_v[D, :]

    def body(blk, carry):
        r = blk * L + iota

        def inner(c, acc):
            for dj in range(5):
                dd = zeros + (c * 5 + dj)
                u = plsc.load_gather(urows_v, [r, dd])
                m = plsc.load_gather(mrows_v, [r, dd])
                wv = plsc.load_gather(w_v, [dd, zeros])
                acc = acc + u * m * wv
            return acc

        acc = lax.fori_loop(0, D // 5, inner, bvec)
        out_v[pl.ds(blk * L, L)] = acc
        return carry

    lax.fori_loop(0, NBLK, body, None)

    pltpu.sync_copy(out_v, out_hbm.at[pl.ds(base, BPW)])


def kernel(train_data, user_emb, movie_emb, W_out, b_out):
    out = _sc_dot(train_data, user_emb, movie_emb, W_out, b_out)
    return out.reshape(BATCH, 1)
